# Initial kernel scaffold; baseline (speedup 1.0000x reference)
import jax
import jax.numpy as jnp
from jax.experimental import pallas as pl


def kernel(input_tensor):
    raise NotImplementedError("write your pallas kernel here")


# speedup vs baseline: 57.2134x; 57.2134x over previous
import functools

import jax
import jax.numpy as jnp
from jax import lax
from jax.experimental import pallas as pl
from jax.experimental.pallas import tpu as pltpu
from jax.experimental.pallas import tpu_sc as plsc

N_ROWS = 1_000_000
N_COLS = 64
TOTAL = N_ROWS * N_COLS
NUM_CORES = 2
NUM_SUBCORES = 16
NW = NUM_CORES * NUM_SUBCORES
PER_W = TOTAL // NW
CHUNK = 40_000
VECS = CHUNK // 16
N_CHUNKS = PER_W // CHUNK

_mesh = plsc.VectorSubcoreMesh(core_axis_name="c", subcore_axis_name="s")


@functools.partial(
    pl.kernel,
    mesh=_mesh,
    out_type=jax.ShapeDtypeStruct((TOTAL,), jnp.float32),
    scratch_types=[pltpu.VMEM((CHUNK,), jnp.float32)],
)
def _mul10(x_hbm, o_hbm, buf):
    wid = lax.axis_index("s") * NUM_CORES + lax.axis_index("c")
    base = wid * PER_W

    def chunk_body(c, carry):
        off = base + c * CHUNK
        pltpu.sync_copy(x_hbm.at[pl.ds(off, CHUNK)], buf)

        def vec_body(i, carry2):
            sl = pl.ds(i * 16, 16)
            buf[sl] = buf[sl] * 10.0
            return carry2

        lax.fori_loop(0, VECS, vec_body, 0)
        pltpu.sync_copy(buf, o_hbm.at[pl.ds(off, CHUNK)])
        return carry

    lax.fori_loop(0, N_CHUNKS, chunk_body, 0)


def kernel(input_tensor):
    flat = input_tensor.reshape(TOTAL)
    out = _mul10(flat)
    return out.reshape(N_ROWS, N_COLS)


# speedup vs baseline: 65.7047x; 1.1484x over previous
import functools

import jax
import jax.numpy as jnp
from jax import lax
from jax.experimental import pallas as pl
from jax.experimental.pallas import tpu as pltpu
from jax.experimental.pallas import tpu_sc as plsc

N_ROWS = 1_000_000
N_COLS = 64
TOTAL = N_ROWS * N_COLS
NUM_CORES = 2
NUM_SUBCORES = 16
NW = NUM_CORES * NUM_SUBCORES
PER_W = TOTAL // NW
CHUNK = 20_000
VECS = CHUNK // 16
N_CHUNKS = PER_W // CHUNK
N_GROUPS = N_CHUNKS // 2
UNROLL = 10

_mesh = plsc.VectorSubcoreMesh(core_axis_name="c", subcore_axis_name="s")


@functools.partial(
    pl.kernel,
    mesh=_mesh,
    out_type=jax.ShapeDtypeStruct((TOTAL,), jnp.float32),
    scratch_types=[
        pltpu.VMEM((CHUNK,), jnp.float32),
        pltpu.VMEM((CHUNK,), jnp.float32),
        pltpu.VMEM((CHUNK,), jnp.float32),
        pltpu.VMEM((CHUNK,), jnp.float32),
        pltpu.SemaphoreType.DMA,
        pltpu.SemaphoreType.DMA,
        pltpu.SemaphoreType.DMA,
        pltpu.SemaphoreType.DMA,
    ],
)
def _mul10(x_hbm, o_hbm, ib0, ib1, ob0, ob1, is0, is1, os0, os1):
    wid = lax.axis_index("s") * NUM_CORES + lax.axis_index("c")
    base = wid * PER_W
    ibufs, obufs = (ib0, ib1), (ob0, ob1)
    isems, osems = (is0, is1), (os0, os1)

    pltpu.async_copy(x_hbm.at[pl.ds(base, CHUNK)], ib0, is0)
    pltpu.async_copy(x_hbm.at[pl.ds(base + CHUNK, CHUNK)], ib1, is1)

    def group_body(g, carry):
        for b in range(2):
            c_off = base + (g * 2 + b) * CHUNK
            ib, ob = ibufs[b], obufs[b]
            pltpu.make_async_copy(x_hbm.at[pl.ds(c_off, CHUNK)], ib, isems[b]).wait()

            @pl.when(g >= 1)
            def _wait_out():
                pltpu.make_async_copy(ob, o_hbm.at[pl.ds(c_off, CHUNK)], osems[b]).wait()

            def vec_body(i, carry2):
                v0 = i * (16 * UNROLL)
                for k in range(UNROLL):
                    sl = pl.ds(v0 + k * 16, 16)
                    ob[sl] = ib[sl] * 10.0
                return carry2

            lax.fori_loop(0, VECS // UNROLL, vec_body, 0)

            pltpu.async_copy(ob, o_hbm.at[pl.ds(c_off, CHUNK)], osems[b])

            @pl.when(g < N_GROUPS - 1)
            def _next_in():
                n_off = c_off + 2 * CHUNK
                pltpu.async_copy(x_hbm.at[pl.ds(n_off, CHUNK)], ib, isems[b])

        return carry

    lax.fori_loop(0, N_GROUPS, group_body, 0)

    tail = base + (N_CHUNKS - 2) * CHUNK
    pltpu.make_async_copy(ob0, o_hbm.at[pl.ds(tail, CHUNK)], os0).wait()
    pltpu.make_async_copy(ob1, o_hbm.at[pl.ds(tail + CHUNK, CHUNK)], os1).wait()


def kernel(input_tensor):
    flat = input_tensor.reshape(TOTAL)
    out = _mul10(flat)
    return out.reshape(N_ROWS, N_COLS)


# speedup vs baseline: 111.8023x; 1.7016x over previous
import functools

import jax
import jax.numpy as jnp
from jax import lax
from jax.experimental import pallas as pl
from jax.experimental.pallas import tpu as pltpu
from jax.experimental.pallas import tpu_sc as plsc

N_ROWS = 1_000_000
N_COLS = 64
NUM_CORES = 2
NUM_SUBCORES = 16
NW = NUM_CORES * NUM_SUBCORES
R = 248
N_CHUNKS_W = 126
ROWS_W = N_CHUNKS_W * R
TAIL0 = NW * ROWS_W
TAIL_R = 8
N_GROUPS = N_CHUNKS_W // 2
ROWS_PER_ITER = 2

_mesh = plsc.VectorSubcoreMesh(core_axis_name="c", subcore_axis_name="s")


def _mul_rows(ib, ob, n_rows):
    def body(i, carry):
        r0 = i * ROWS_PER_ITER
        for dr in range(ROWS_PER_ITER):
            for k in range(N_COLS // 16):
                sl = pl.ds(k * 16, 16)
                ob[r0 + dr, sl] = ib[r0 + dr, sl] * 10.0
        return carry

    lax.fori_loop(0, n_rows // ROWS_PER_ITER, body, 0)


@functools.partial(
    pl.kernel,
    mesh=_mesh,
    out_type=jax.ShapeDtypeStruct((N_ROWS, N_COLS), jnp.float32),
    scratch_types=[
        pltpu.VMEM((R, N_COLS), jnp.float32),
        pltpu.VMEM((R, N_COLS), jnp.float32),
        pltpu.VMEM((R, N_COLS), jnp.float32),
        pltpu.VMEM((R, N_COLS), jnp.float32),
        pltpu.SemaphoreType.DMA,
        pltpu.SemaphoreType.DMA,
        pltpu.SemaphoreType.DMA,
        pltpu.SemaphoreType.DMA,
    ],
)
def _mul10(x_hbm, o_hbm, ib0, ib1, ob0, ob1, is0, is1, os0, os1):
    wid = lax.axis_index("s") * NUM_CORES + lax.axis_index("c")
    base = pl.multiple_of(wid * ROWS_W, 8)
    ibufs, obufs = (ib0, ib1), (ob0, ob1)
    isems, osems = (is0, is1), (os0, os1)

    def row0_of(c):
        return pl.multiple_of(base + c * R, 8)

    pltpu.async_copy(x_hbm.at[pl.ds(row0_of(0), R), :], ib0, is0)
    pltpu.async_copy(x_hbm.at[pl.ds(row0_of(1), R), :], ib1, is1)

    def group_body(g, carry):
        for b in range(2):
            c = g * 2 + b
            row0 = row0_of(c)
            ib, ob = ibufs[b], obufs[b]
            pltpu.make_async_copy(x_hbm.at[pl.ds(row0, R), :], ib, isems[b]).wait()

            @pl.when(g >= 1)
            def _wait_out():
                pltpu.make_async_copy(ob, o_hbm.at[pl.ds(row0, R), :], osems[b]).wait()

            _mul_rows(ib, ob, R)

            pltpu.async_copy(ob, o_hbm.at[pl.ds(row0, R), :], osems[b])

            @pl.when(g < N_GROUPS - 1)
            def _next_in():
                pltpu.async_copy(
                    x_hbm.at[pl.ds(row0_of(c + 2), R), :], ib, isems[b]
                )

        return carry

    lax.fori_loop(0, N_GROUPS, group_body, 0)

    tail_row = pl.multiple_of(TAIL0 + wid * TAIL_R, 8)
    is_tail_w = wid < 8

    @pl.when(is_tail_w)
    def _tail_in():
        pltpu.async_copy(
            x_hbm.at[pl.ds(tail_row, TAIL_R), :], ib0.at[pl.ds(0, TAIL_R), :], is0
        )

    rl0, rl1 = row0_of(N_CHUNKS_W - 2), row0_of(N_CHUNKS_W - 1)
    pltpu.make_async_copy(ob0, o_hbm.at[pl.ds(rl0, R), :], os0).wait()
    pltpu.make_async_copy(ob1, o_hbm.at[pl.ds(rl1, R), :], os1).wait()

    @pl.when(is_tail_w)
    def _tail_work():
        pltpu.make_async_copy(
            x_hbm.at[pl.ds(tail_row, TAIL_R), :], ib0.at[pl.ds(0, TAIL_R), :], is0
        ).wait()
        _mul_rows(ib0, ob0, TAIL_R)
        pltpu.async_copy(
            ob0.at[pl.ds(0, TAIL_R), :], o_hbm.at[pl.ds(tail_row, TAIL_R), :], os0
        )
        pltpu.make_async_copy(
            ob0.at[pl.ds(0, TAIL_R), :], o_hbm.at[pl.ds(tail_row, TAIL_R), :], os0
        ).wait()


def kernel(input_tensor):
    return _mul10(input_tensor)
